# baseline (speedup 1.0000x reference)
import functools

import jax
import jax.numpy as jnp
from jax import lax
from jax.experimental import pallas as pl
from jax.experimental.pallas import tpu as pltpu
from jax.experimental.pallas import tpu_sc as plsc

N = 10000
E = 320000
F_IN = 128
HID = 64
C = 16

NC = 2
NS = 16
NW = NC * NS
EW = E // NW
K = 125
NCH = EW // K
NBUF = 4
KD = 80
NCHD = EW // KD
NP = 10240
RPT = NP // NS
ZR = 128
DEG_D = 16

_SC_PARAMS = pltpu.CompilerParams(use_tc_tiling_on_sc=False)


@functools.cache
def _get_mesh():
    return plsc.VectorSubcoreMesh(
        core_axis_name="c", subcore_axis_name="s", num_cores=NC, num_subcores=NS
    )


def _fill_rows16(ref, rows, val):
    v = jnp.full((16,), val, jnp.float32)

    def body(i, carry):
        ref[i] = v
        return carry

    lax.fori_loop(0, rows, body, 0)


def _zero2d(ref, rows, d):
    z = jnp.zeros((16,), jnp.float32)

    def body(i, carry):
        for c0 in range(d // 16):
            ref[i, pl.ds(c0 * 16, 16)] = z
        return carry

    lax.fori_loop(0, rows, body, 0)


def _stage_idx(ei_hbm, row, ebase, idx_ref, sem):
    def fire(j, carry):
        pltpu.make_async_copy(
            ei_hbm.at[row, pl.ds(ebase + j * KD, KD)], idx_ref.at[j], sem
        ).start()
        return carry

    lax.fori_loop(0, NCHD, fire, 0)

    def drain(j, carry):
        pltpu.make_async_copy(
            ei_hbm.at[row, pl.ds(ebase + j * KD, KD)], idx_ref.at[j], sem
        ).wait()
        return carry

    lax.fori_loop(0, NCHD, drain, 0)


def _writeout(acc, zbuf, base_r, out_ref):
    def body(i, carry):
        r0 = base_r + i * ZR
        pltpu.sync_copy(acc.at[pl.ds(r0, ZR)], zbuf)
        pltpu.sync_copy(zbuf, out_ref.at[pl.ds(r0, ZR)])
        return carry

    lax.fori_loop(0, RPT // ZR, body, 0)


@functools.cache
def _make_sc_degree():
    @functools.partial(
        pl.kernel,
        out_type=(
            jax.ShapeDtypeStruct((NP, DEG_D), jnp.float32),
            jax.ShapeDtypeStruct((NP, DEG_D), jnp.float32),
        ),
        mesh=_get_mesh(),
        scratch_types=[
            pltpu.VMEM((NCHD, KD), jnp.int32),
            pltpu.VMEM((KD, DEG_D), jnp.float32),
            pltpu.VMEM((ZR, DEG_D), jnp.float32),
            pltpu.VMEM_SHARED((NP, DEG_D), jnp.float32),
            pltpu.SemaphoreType.DMA,
        ],
        name="sc_gcn_degree",
        compiler_params=_SC_PARAMS,
    )
    def _sc_degree(ei_hbm, out0, out1, idx_d, ones_b, zbuf, acc, sem):
        cid = lax.axis_index("c")
        sid = lax.axis_index("s")
        wid = cid * NS + sid
        _stage_idx(ei_hbm, 1, wid * EW, idx_d, sem)
        _fill_rows16(ones_b, KD, 1.0)
        _fill_rows16(zbuf, ZR, 0.0)
        base_r = sid * RPT

        def zc(i, carry):
            pltpu.sync_copy(zbuf, acc.at[pl.ds(base_r + i * ZR, ZR)])
            return carry

        lax.fori_loop(0, RPT // ZR, zc, 0)
        plsc.subcore_barrier()

        def ch(j, carry):
            pltpu.async_copy(ones_b, acc.at[idx_d.at[j]], sem, add=True)
            return carry

        lax.fori_loop(0, NCHD, ch, 0)

        def chw(j, carry):
            pltpu.make_async_copy(ones_b, acc.at[idx_d.at[j]], sem).wait()
            return carry

        lax.fori_loop(0, NCHD, chw, 0)
        plsc.subcore_barrier()

        @pl.when(cid == 0)
        def _():
            _writeout(acc, zbuf, base_r, out0)

        @pl.when(cid == 1)
        def _():
            _writeout(acc, zbuf, base_r, out1)

    return _sc_degree


@functools.cache
def _make_sc_agg(d, tab_bufs):
    scratch = [
        pltpu.VMEM((NCH, K), jnp.int32),
        pltpu.VMEM((NCH, K), jnp.int32),
    ]
    scratch += [pltpu.VMEM((K, d), jnp.float32) for _ in range(NBUF)]
    scratch += [
        pltpu.VMEM((ZR, d), jnp.float32),
        pltpu.VMEM_SHARED((NP, d), jnp.float32),
    ]
    scratch += [pltpu.SemaphoreType.DMA for _ in range(2 * NBUF)]
    if tab_bufs:
        scratch.append(pltpu.VMEM_SHARED((NP, d), jnp.float32))

    @functools.partial(
        pl.kernel,
        out_type=(
            jax.ShapeDtypeStruct((NP, d), jnp.float32),
            jax.ShapeDtypeStruct((NP, d), jnp.float32),
        ),
        mesh=_get_mesh(),
        scratch_types=scratch,
        name=f"sc_gcn_agg_{d}",
        compiler_params=_SC_PARAMS,
    )
    def _agg(y_hbm, src_hbm, dst_hbm, out0, out1, idx_s, idx_d,
             rows0, rows1, rows2, rows3, zbuf, acc,
             sem0, sem1, sem2, sem3, ssem0, ssem1, ssem2, ssem3, *maybe_tab):
        cid = lax.axis_index("c")
        sid = lax.axis_index("s")
        wid = cid * NS + sid
        pltpu.sync_copy(src_hbm.at[wid], idx_s)
        pltpu.sync_copy(dst_hbm.at[wid], idx_d)
        base_r = sid * RPT

        if tab_bufs:
            ytab = maybe_tab[0]

            def st(i, carry):
                r0 = base_r + i * ZR
                pltpu.sync_copy(y_hbm.at[pl.ds(r0, ZR)], zbuf)
                pltpu.sync_copy(zbuf, ytab.at[pl.ds(r0, ZR)])
                return carry

            lax.fori_loop(0, RPT // ZR, st, 0)

        gsrcs = tuple(
            (maybe_tab[0] if b in tab_bufs else y_hbm) for b in range(NBUF)
        )

        _zero2d(zbuf, ZR, d)

        def zc(i, carry):
            pltpu.sync_copy(zbuf, acc.at[pl.ds(base_r + i * ZR, ZR)])
            return carry

        lax.fori_loop(0, RPT // ZR, zc, 0)
        plsc.subcore_barrier()

        rows = (rows0, rows1, rows2, rows3)
        sems = (sem0, sem1, sem2, sem3)
        ssems = (ssem0, ssem1, ssem2, ssem3)

        def g_start(j, b):
            pltpu.make_async_copy(gsrcs[b].at[idx_s.at[j]], rows[b], sems[b]).start()

        def g_wait(j, b):
            pltpu.make_async_copy(gsrcs[b].at[idx_s.at[j]], rows[b], sems[b]).wait()

        def sc_fire(j, b):
            pltpu.async_copy(rows[b], acc.at[idx_d.at[j]], ssems[b], add=True)

        def sc_wait(j, b):
            pltpu.make_async_copy(rows[b], acc.at[idx_d.at[j]], ssems[b]).wait()

        for b in range(NBUF):
            g_start(b, b)

        def body(i, carry):
            for b in range(NBUF):
                j = NBUF * i + b
                g_wait(j, b)
                sc_fire(j, b)
            for b in range(NBUF):
                j = NBUF * i + b
                sc_wait(j, b)
                g_start(j + NBUF, b)
            return carry

        lax.fori_loop(0, NCH // NBUF - 1, body, 0)

        for b in range(NBUF):
            j = NCH - NBUF + b
            g_wait(j, b)
            sc_fire(j, b)
        for b in range(NBUF):
            j = NCH - NBUF + b
            sc_wait(j, b)

        plsc.subcore_barrier()

        @pl.when(cid == 0)
        def _():
            _writeout(acc, zbuf, base_r, out0)

        @pl.when(cid == 1)
        def _():
            _writeout(acc, zbuf, base_r, out1)

    return _agg


_R = 5120
_GRID = NP // _R


def _dinv_col(d0, d1):
    deg = d0[:, 0:1] + d1[:, 0:1] + 1.0
    return lax.rsqrt(deg)


def _tc1_body(x_ref, w_ref, d0_ref, d1_ref, y_ref):
    dinv = _dinv_col(d0_ref[...], d1_ref[...])
    xw = jnp.dot(x_ref[...], w_ref[...], preferred_element_type=jnp.float32)
    y_ref[...] = xw * dinv


def _tc_phase1(x, W1, d0, d1):
    return pl.pallas_call(
        _tc1_body,
        grid=(_GRID,),
        in_specs=[
            pl.BlockSpec((_R, F_IN), lambda i: (i, 0)),
            pl.BlockSpec((F_IN, HID), lambda i: (0, 0)),
            pl.BlockSpec((_R, DEG_D), lambda i: (i, 0)),
            pl.BlockSpec((_R, DEG_D), lambda i: (i, 0)),
        ],
        out_specs=pl.BlockSpec((_R, HID), lambda i: (i, 0)),
        out_shape=jax.ShapeDtypeStruct((NP, HID), jnp.float32),
    )(x, W1, d0, d1)


def _tc2_body(p0_ref, p1_ref, y1_ref, d0_ref, d1_ref, b1_ref, w2_ref, y2_ref):
    dinv = _dinv_col(d0_ref[...], d1_ref[...])
    h = dinv * (p0_ref[...] + p1_ref[...] + y1_ref[...]) + b1_ref[...]
    h = jnp.maximum(h, 0.0)
    xw2 = jnp.dot(h, w2_ref[...], preferred_element_type=jnp.float32)
    y2_ref[...] = xw2 * dinv


def _tc_phase2(p0, p1, y1, d0, d1, b1, W2):
    return pl.pallas_call(
        _tc2_body,
        grid=(_GRID,),
        in_specs=[
            pl.BlockSpec((_R, HID), lambda i: (i, 0)),
            pl.BlockSpec((_R, HID), lambda i: (i, 0)),
            pl.BlockSpec((_R, HID), lambda i: (i, 0)),
            pl.BlockSpec((_R, DEG_D), lambda i: (i, 0)),
            pl.BlockSpec((_R, DEG_D), lambda i: (i, 0)),
            pl.BlockSpec((1, HID), lambda i: (0, 0)),
            pl.BlockSpec((HID, C), lambda i: (0, 0)),
        ],
        out_specs=pl.BlockSpec((_R, C), lambda i: (i, 0)),
        out_shape=jax.ShapeDtypeStruct((NP, C), jnp.float32),
    )(p0, p1, y1, d0, d1, b1, W2)


def _tc3_body(q0_ref, q1_ref, y2_ref, d0_ref, d1_ref, b2_ref, o_ref):
    dinv = _dinv_col(d0_ref[...], d1_ref[...])
    o = dinv * (q0_ref[...] + q1_ref[...] + y2_ref[...]) + b2_ref[...]
    m = jnp.max(o, axis=1, keepdims=True)
    e = jnp.exp(o - m)
    s = jnp.sum(e, axis=1, keepdims=True)
    o_ref[...] = o - m - jnp.log(s)


def _tc_phase3(q0, q1, y2, d0, d1, b2):
    return pl.pallas_call(
        _tc3_body,
        grid=(_GRID,),
        in_specs=[
            pl.BlockSpec((_R, C), lambda i: (i, 0)),
            pl.BlockSpec((_R, C), lambda i: (i, 0)),
            pl.BlockSpec((_R, C), lambda i: (i, 0)),
            pl.BlockSpec((_R, DEG_D), lambda i: (i, 0)),
            pl.BlockSpec((_R, DEG_D), lambda i: (i, 0)),
            pl.BlockSpec((1, C), lambda i: (0, 0)),
        ],
        out_specs=pl.BlockSpec((_R, C), lambda i: (i, 0)),
        out_shape=jax.ShapeDtypeStruct((N, C), jnp.float32),
    )(q0, q1, y2, d0, d1, b2)


def kernel(x, edge_index, W1, b1, W2, b2):
    src3 = edge_index[0].reshape(NW, NCH, K)
    dst3 = edge_index[1].reshape(NW, NCH, K)
    d0, d1 = _make_sc_degree()(edge_index)
    y1 = _tc_phase1(x, W1, d0, d1)
    p0, p1 = _make_sc_agg(HID, ())(y1, src3, dst3)
    y2 = _tc_phase2(p0, p1, y1, d0, d1, b1.reshape(1, HID), W2)
    q0, q1 = _make_sc_agg(C, (0, 1, 2, 3))(y2, src3, dst3)
    return _tc_phase3(q0, q1, y2, d0, d1, b2.reshape(1, C))


# speedup vs baseline: 1.0607x; 1.0607x over previous
import functools

import jax
import jax.numpy as jnp
from jax import lax
from jax.experimental import pallas as pl
from jax.experimental.pallas import tpu as pltpu
from jax.experimental.pallas import tpu_sc as plsc

N = 10000
E = 320000
F_IN = 128
HID = 64
C = 16

NC = 2
NS = 16
NW = NC * NS
EW = E // NW
K = 125
NCH = EW // K
NBUF = 4
KD = 80
NCHD = EW // KD
NP = 10240
RPT = NP // NS
ZR = 128
DEG_D = 16

_SC_PARAMS = pltpu.CompilerParams(use_tc_tiling_on_sc=False)


@functools.cache
def _get_mesh():
    return plsc.VectorSubcoreMesh(
        core_axis_name="c", subcore_axis_name="s", num_cores=NC, num_subcores=NS
    )


def _fill_rows16(ref, rows, val):
    v = jnp.full((16,), val, jnp.float32)

    def body(i, carry):
        ref[i] = v
        return carry

    lax.fori_loop(0, rows, body, 0)


def _zero2d(ref, rows, d):
    z = jnp.zeros((16,), jnp.float32)

    def body(i, carry):
        for c0 in range(d // 16):
            ref[i, pl.ds(c0 * 16, 16)] = z
        return carry

    lax.fori_loop(0, rows, body, 0)


def _stage_idx(ei_hbm, row, ebase, idx_ref, sem):
    def fire(j, carry):
        pltpu.make_async_copy(
            ei_hbm.at[row, pl.ds(ebase + j * KD, KD)], idx_ref.at[j], sem
        ).start()
        return carry

    lax.fori_loop(0, NCHD, fire, 0)

    def drain(j, carry):
        pltpu.make_async_copy(
            ei_hbm.at[row, pl.ds(ebase + j * KD, KD)], idx_ref.at[j], sem
        ).wait()
        return carry

    lax.fori_loop(0, NCHD, drain, 0)


def _writeout(acc, zbuf, base_r, out_ref):
    def body(i, carry):
        r0 = base_r + i * ZR
        pltpu.sync_copy(acc.at[pl.ds(r0, ZR)], zbuf)
        pltpu.sync_copy(zbuf, out_ref.at[pl.ds(r0, ZR)])
        return carry

    lax.fori_loop(0, RPT // ZR, body, 0)


@functools.cache
def _make_sc_degree():
    @functools.partial(
        pl.kernel,
        out_type=(
            jax.ShapeDtypeStruct((NP, DEG_D), jnp.float32),
            jax.ShapeDtypeStruct((NP, DEG_D), jnp.float32),
        ),
        mesh=_get_mesh(),
        scratch_types=[
            pltpu.VMEM((NCHD, KD), jnp.int32),
            pltpu.VMEM((KD, DEG_D), jnp.float32),
            pltpu.VMEM((ZR, DEG_D), jnp.float32),
            pltpu.VMEM_SHARED((NP, DEG_D), jnp.float32),
            pltpu.SemaphoreType.DMA,
        ],
        name="sc_gcn_degree",
        compiler_params=_SC_PARAMS,
    )
    def _sc_degree(ei_hbm, out0, out1, idx_d, ones_b, zbuf, acc, sem):
        cid = lax.axis_index("c")
        sid = lax.axis_index("s")
        wid = cid * NS + sid
        _stage_idx(ei_hbm, 1, wid * EW, idx_d, sem)
        _fill_rows16(ones_b, KD, 1.0)
        _fill_rows16(zbuf, ZR, 0.0)
        base_r = sid * RPT

        def zc(i, carry):
            pltpu.sync_copy(zbuf, acc.at[pl.ds(base_r + i * ZR, ZR)])
            return carry

        lax.fori_loop(0, RPT // ZR, zc, 0)
        plsc.subcore_barrier()

        def ch(j, carry):
            pltpu.async_copy(ones_b, acc.at[idx_d.at[j]], sem, add=True)
            return carry

        lax.fori_loop(0, NCHD, ch, 0)

        def chw(j, carry):
            pltpu.make_async_copy(ones_b, acc.at[idx_d.at[j]], sem).wait()
            return carry

        lax.fori_loop(0, NCHD, chw, 0)
        plsc.subcore_barrier()

        @pl.when(cid == 0)
        def _():
            _writeout(acc, zbuf, base_r, out0)

        @pl.when(cid == 1)
        def _():
            _writeout(acc, zbuf, base_r, out1)

    return _sc_degree


@functools.cache
def _make_sc_agg(d, tab_bufs):
    scratch = [
        pltpu.VMEM((NCH, K), jnp.int32),
        pltpu.VMEM((NCH, K), jnp.int32),
    ]
    scratch += [pltpu.VMEM((K, d), jnp.float32) for _ in range(NBUF)]
    scratch += [
        pltpu.VMEM((ZR, d), jnp.float32),
        pltpu.VMEM_SHARED((NP, d), jnp.float32),
    ]
    scratch += [pltpu.SemaphoreType.DMA for _ in range(2 * NBUF)]
    if tab_bufs:
        scratch.append(pltpu.VMEM_SHARED((NP, d), jnp.float32))

    @functools.partial(
        pl.kernel,
        out_type=(
            jax.ShapeDtypeStruct((NP, d), jnp.float32),
            jax.ShapeDtypeStruct((NP, d), jnp.float32),
        ),
        mesh=_get_mesh(),
        scratch_types=scratch,
        name=f"sc_gcn_agg_{d}",
        compiler_params=_SC_PARAMS,
    )
    def _agg(y_hbm, src_hbm, dst_hbm, out0, out1, idx_s, idx_d,
             rows0, rows1, rows2, rows3, zbuf, acc,
             sem0, sem1, sem2, sem3, ssem0, ssem1, ssem2, ssem3, *maybe_tab):
        cid = lax.axis_index("c")
        sid = lax.axis_index("s")
        wid = cid * NS + sid
        pltpu.sync_copy(src_hbm.at[wid], idx_s)
        pltpu.sync_copy(dst_hbm.at[wid], idx_d)
        base_r = sid * RPT

        if tab_bufs:
            ytab = maybe_tab[0]

            def st(i, carry):
                r0 = base_r + i * ZR
                pltpu.sync_copy(y_hbm.at[pl.ds(r0, ZR)], zbuf)
                pltpu.sync_copy(zbuf, ytab.at[pl.ds(r0, ZR)])
                return carry

            lax.fori_loop(0, RPT // ZR, st, 0)

        gsrcs = tuple(
            (maybe_tab[0] if b in tab_bufs else y_hbm) for b in range(NBUF)
        )

        _zero2d(zbuf, ZR, d)

        def zc(i, carry):
            pltpu.sync_copy(zbuf, acc.at[pl.ds(base_r + i * ZR, ZR)])
            return carry

        lax.fori_loop(0, RPT // ZR, zc, 0)
        plsc.subcore_barrier()

        rows = (rows0, rows1, rows2, rows3)
        sems = (sem0, sem1, sem2, sem3)
        ssems = (ssem0, ssem1, ssem2, ssem3)

        def g_start(j, b):
            pltpu.make_async_copy(gsrcs[b].at[idx_s.at[j]], rows[b], sems[b]).start()

        def g_wait(j, b):
            pltpu.make_async_copy(gsrcs[b].at[idx_s.at[j]], rows[b], sems[b]).wait()

        def scat(j, b):
            pltpu.sync_copy(rows[b], acc.at[idx_d.at[j]], add=True)

        for b in range(NBUF):
            g_start(b, b)

        def body(i, carry):
            for b in range(NBUF):
                j = NBUF * i + b
                g_wait(j, b)
                scat(j, b)
                g_start(j + NBUF, b)
            return carry

        lax.fori_loop(0, NCH // NBUF - 1, body, 0)

        for b in range(NBUF):
            j = NCH - NBUF + b
            g_wait(j, b)
            scat(j, b)

        plsc.subcore_barrier()

        @pl.when(cid == 0)
        def _():
            _writeout(acc, zbuf, base_r, out0)

        @pl.when(cid == 1)
        def _():
            _writeout(acc, zbuf, base_r, out1)

    return _agg


_R = 5120
_GRID = NP // _R


def _dinv_col(d0, d1):
    deg = d0[:, 0:1] + d1[:, 0:1] + 1.0
    return lax.rsqrt(deg)


def _tc1_body(x_ref, w_ref, d0_ref, d1_ref, y_ref):
    dinv = _dinv_col(d0_ref[...], d1_ref[...])
    xw = jnp.dot(x_ref[...], w_ref[...], preferred_element_type=jnp.float32)
    y_ref[...] = xw * dinv


def _tc_phase1(x, W1, d0, d1):
    return pl.pallas_call(
        _tc1_body,
        grid=(_GRID,),
        in_specs=[
            pl.BlockSpec((_R, F_IN), lambda i: (i, 0)),
            pl.BlockSpec((F_IN, HID), lambda i: (0, 0)),
            pl.BlockSpec((_R, DEG_D), lambda i: (i, 0)),
            pl.BlockSpec((_R, DEG_D), lambda i: (i, 0)),
        ],
        out_specs=pl.BlockSpec((_R, HID), lambda i: (i, 0)),
        out_shape=jax.ShapeDtypeStruct((NP, HID), jnp.float32),
    )(x, W1, d0, d1)


def _tc2_body(p0_ref, p1_ref, y1_ref, d0_ref, d1_ref, b1_ref, w2_ref, y2_ref):
    dinv = _dinv_col(d0_ref[...], d1_ref[...])
    h = dinv * (p0_ref[...] + p1_ref[...] + y1_ref[...]) + b1_ref[...]
    h = jnp.maximum(h, 0.0)
    xw2 = jnp.dot(h, w2_ref[...], preferred_element_type=jnp.float32)
    y2_ref[...] = xw2 * dinv


def _tc_phase2(p0, p1, y1, d0, d1, b1, W2):
    return pl.pallas_call(
        _tc2_body,
        grid=(_GRID,),
        in_specs=[
            pl.BlockSpec((_R, HID), lambda i: (i, 0)),
            pl.BlockSpec((_R, HID), lambda i: (i, 0)),
            pl.BlockSpec((_R, HID), lambda i: (i, 0)),
            pl.BlockSpec((_R, DEG_D), lambda i: (i, 0)),
            pl.BlockSpec((_R, DEG_D), lambda i: (i, 0)),
            pl.BlockSpec((1, HID), lambda i: (0, 0)),
            pl.BlockSpec((HID, C), lambda i: (0, 0)),
        ],
        out_specs=pl.BlockSpec((_R, C), lambda i: (i, 0)),
        out_shape=jax.ShapeDtypeStruct((NP, C), jnp.float32),
    )(p0, p1, y1, d0, d1, b1, W2)


def _tc3_body(q0_ref, q1_ref, y2_ref, d0_ref, d1_ref, b2_ref, o_ref):
    dinv = _dinv_col(d0_ref[...], d1_ref[...])
    o = dinv * (q0_ref[...] + q1_ref[...] + y2_ref[...]) + b2_ref[...]
    m = jnp.max(o, axis=1, keepdims=True)
    e = jnp.exp(o - m)
    s = jnp.sum(e, axis=1, keepdims=True)
    o_ref[...] = o - m - jnp.log(s)


def _tc_phase3(q0, q1, y2, d0, d1, b2):
    return pl.pallas_call(
        _tc3_body,
        grid=(_GRID,),
        in_specs=[
            pl.BlockSpec((_R, C), lambda i: (i, 0)),
            pl.BlockSpec((_R, C), lambda i: (i, 0)),
            pl.BlockSpec((_R, C), lambda i: (i, 0)),
            pl.BlockSpec((_R, DEG_D), lambda i: (i, 0)),
            pl.BlockSpec((_R, DEG_D), lambda i: (i, 0)),
            pl.BlockSpec((1, C), lambda i: (0, 0)),
        ],
        out_specs=pl.BlockSpec((_R, C), lambda i: (i, 0)),
        out_shape=jax.ShapeDtypeStruct((N, C), jnp.float32),
    )(q0, q1, y2, d0, d1, b2)


def kernel(x, edge_index, W1, b1, W2, b2):
    src3 = edge_index[0].reshape(NW, NCH, K)
    dst3 = edge_index[1].reshape(NW, NCH, K)
    d0, d1 = _make_sc_degree()(edge_index)
    y1 = _tc_phase1(x, W1, d0, d1)
    p0, p1 = _make_sc_agg(HID, ())(y1, src3, dst3)
    y2 = _tc_phase2(p0, p1, y1, d0, d1, b1.reshape(1, HID), W2)
    q0, q1 = _make_sc_agg(C, (0, 1, 2, 3))(y2, src3, dst3)
    return _tc_phase3(q0, q1, y2, d0, d1, b2.reshape(1, C))


# speedup vs baseline: 1.0634x; 1.0026x over previous
import functools

import jax
import jax.numpy as jnp
from jax import lax
from jax.experimental import pallas as pl
from jax.experimental.pallas import tpu as pltpu
from jax.experimental.pallas import tpu_sc as plsc

N = 10000
E = 320000
F_IN = 128
HID = 64
C = 16

NC = 2
NS = 16
NW = NC * NS
EW = E // NW
K = 125
NCH = EW // K
NBUF = 5
KD = 80
NCHD = EW // KD
NP = 10240
RPT = NP // NS
ZR = 128
DEG_D = 16

_SC_PARAMS = pltpu.CompilerParams(use_tc_tiling_on_sc=False)


@functools.cache
def _get_mesh():
    return plsc.VectorSubcoreMesh(
        core_axis_name="c", subcore_axis_name="s", num_cores=NC, num_subcores=NS
    )


def _fill_rows16(ref, rows, val):
    v = jnp.full((16,), val, jnp.float32)

    def body(i, carry):
        ref[i] = v
        return carry

    lax.fori_loop(0, rows, body, 0)


def _zero2d(ref, rows, d):
    z = jnp.zeros((16,), jnp.float32)

    def body(i, carry):
        for c0 in range(d // 16):
            ref[i, pl.ds(c0 * 16, 16)] = z
        return carry

    lax.fori_loop(0, rows, body, 0)


def _stage_idx(ei_hbm, row, ebase, idx_ref, sem):
    def fire(j, carry):
        pltpu.make_async_copy(
            ei_hbm.at[row, pl.ds(ebase + j * KD, KD)], idx_ref.at[j], sem
        ).start()
        return carry

    lax.fori_loop(0, NCHD, fire, 0)

    def drain(j, carry):
        pltpu.make_async_copy(
            ei_hbm.at[row, pl.ds(ebase + j * KD, KD)], idx_ref.at[j], sem
        ).wait()
        return carry

    lax.fori_loop(0, NCHD, drain, 0)


def _writeout(acc, zbuf, base_r, out_ref):
    def body(i, carry):
        r0 = base_r + i * ZR
        pltpu.sync_copy(acc.at[pl.ds(r0, ZR)], zbuf)
        pltpu.sync_copy(zbuf, out_ref.at[pl.ds(r0, ZR)])
        return carry

    lax.fori_loop(0, RPT // ZR, body, 0)


@functools.cache
def _make_sc_degree():
    @functools.partial(
        pl.kernel,
        out_type=(
            jax.ShapeDtypeStruct((NP, DEG_D), jnp.float32),
            jax.ShapeDtypeStruct((NP, DEG_D), jnp.float32),
        ),
        mesh=_get_mesh(),
        scratch_types=[
            pltpu.VMEM((NCHD, KD), jnp.int32),
            pltpu.VMEM((KD, DEG_D), jnp.float32),
            pltpu.VMEM((ZR, DEG_D), jnp.float32),
            pltpu.VMEM_SHARED((NP, DEG_D), jnp.float32),
            pltpu.SemaphoreType.DMA,
        ],
        name="sc_gcn_degree",
        compiler_params=_SC_PARAMS,
    )
    def _sc_degree(ei_hbm, out0, out1, idx_d, ones_b, zbuf, acc, sem):
        cid = lax.axis_index("c")
        sid = lax.axis_index("s")
        wid = cid * NS + sid
        _stage_idx(ei_hbm, 1, wid * EW, idx_d, sem)
        _fill_rows16(ones_b, KD, 1.0)
        _fill_rows16(zbuf, ZR, 0.0)
        base_r = sid * RPT

        def zc(i, carry):
            pltpu.sync_copy(zbuf, acc.at[pl.ds(base_r + i * ZR, ZR)])
            return carry

        lax.fori_loop(0, RPT // ZR, zc, 0)
        plsc.subcore_barrier()

        def ch(j, carry):
            pltpu.async_copy(ones_b, acc.at[idx_d.at[j]], sem, add=True)
            return carry

        lax.fori_loop(0, NCHD, ch, 0)

        def chw(j, carry):
            pltpu.make_async_copy(ones_b, acc.at[idx_d.at[j]], sem).wait()
            return carry

        lax.fori_loop(0, NCHD, chw, 0)
        plsc.subcore_barrier()

        @pl.when(cid == 0)
        def _():
            _writeout(acc, zbuf, base_r, out0)

        @pl.when(cid == 1)
        def _():
            _writeout(acc, zbuf, base_r, out1)

    return _sc_degree


@functools.cache
def _make_sc_agg(d, tab_bufs):
    scratch = [
        pltpu.VMEM((NCH, K), jnp.int32),
        pltpu.VMEM((NCH, K), jnp.int32),
    ]
    scratch += [pltpu.VMEM((K, d), jnp.float32) for _ in range(NBUF)]
    scratch += [
        pltpu.VMEM((ZR, d), jnp.float32),
        pltpu.VMEM_SHARED((NP, d), jnp.float32),
    ]
    scratch += [pltpu.SemaphoreType.DMA for _ in range(NBUF)]
    if tab_bufs:
        scratch.append(pltpu.VMEM_SHARED((NP, d), jnp.float32))

    @functools.partial(
        pl.kernel,
        out_type=(
            jax.ShapeDtypeStruct((NP, d), jnp.float32),
            jax.ShapeDtypeStruct((NP, d), jnp.float32),
        ),
        mesh=_get_mesh(),
        scratch_types=scratch,
        name=f"sc_gcn_agg_{d}",
        compiler_params=_SC_PARAMS,
    )
    def _agg(y_hbm, src_hbm, dst_hbm, out0, out1, idx_s, idx_d,
             rows0, rows1, rows2, rows3, rows4, zbuf, acc,
             sem0, sem1, sem2, sem3, sem4, *maybe_tab):
        cid = lax.axis_index("c")
        sid = lax.axis_index("s")
        wid = cid * NS + sid
        pltpu.sync_copy(src_hbm.at[wid], idx_s)
        pltpu.sync_copy(dst_hbm.at[wid], idx_d)
        base_r = sid * RPT

        if tab_bufs:
            ytab = maybe_tab[0]

            def st(i, carry):
                r0 = base_r + i * ZR
                pltpu.sync_copy(y_hbm.at[pl.ds(r0, ZR)], zbuf)
                pltpu.sync_copy(zbuf, ytab.at[pl.ds(r0, ZR)])
                return carry

            lax.fori_loop(0, RPT // ZR, st, 0)

        gsrcs = tuple(
            (maybe_tab[0] if b in tab_bufs else y_hbm) for b in range(NBUF)
        )

        _zero2d(zbuf, ZR, d)

        def zc(i, carry):
            pltpu.sync_copy(zbuf, acc.at[pl.ds(base_r + i * ZR, ZR)])
            return carry

        lax.fori_loop(0, RPT // ZR, zc, 0)
        plsc.subcore_barrier()

        rows = (rows0, rows1, rows2, rows3, rows4)
        sems = (sem0, sem1, sem2, sem3, sem4)

        def g_start(j, b):
            pltpu.make_async_copy(gsrcs[b].at[idx_s.at[j]], rows[b], sems[b]).start()

        def g_wait(j, b):
            pltpu.make_async_copy(gsrcs[b].at[idx_s.at[j]], rows[b], sems[b]).wait()

        def scat(j, b):
            pltpu.sync_copy(rows[b], acc.at[idx_d.at[j]], add=True)

        for b in range(NBUF):
            g_start(b, b)

        def body(i, carry):
            for b in range(NBUF):
                j = NBUF * i + b
                g_wait(j, b)
                scat(j, b)
                g_start(j + NBUF, b)
            return carry

        lax.fori_loop(0, NCH // NBUF - 1, body, 0)

        for b in range(NBUF):
            j = NCH - NBUF + b
            g_wait(j, b)
            scat(j, b)

        plsc.subcore_barrier()

        @pl.when(cid == 0)
        def _():
            _writeout(acc, zbuf, base_r, out0)

        @pl.when(cid == 1)
        def _():
            _writeout(acc, zbuf, base_r, out1)

    return _agg


_R = 5120
_GRID = NP // _R


def _dinv_col(d0, d1):
    deg = d0[:, 0:1] + d1[:, 0:1] + 1.0
    return lax.rsqrt(deg)


def _tc1_body(x_ref, w_ref, d0_ref, d1_ref, y_ref):
    dinv = _dinv_col(d0_ref[...], d1_ref[...])
    xw = jnp.dot(x_ref[...], w_ref[...], preferred_element_type=jnp.float32)
    y_ref[...] = xw * dinv


def _tc_phase1(x, W1, d0, d1):
    return pl.pallas_call(
        _tc1_body,
        grid=(_GRID,),
        in_specs=[
            pl.BlockSpec((_R, F_IN), lambda i: (i, 0)),
            pl.BlockSpec((F_IN, HID), lambda i: (0, 0)),
            pl.BlockSpec((_R, DEG_D), lambda i: (i, 0)),
            pl.BlockSpec((_R, DEG_D), lambda i: (i, 0)),
        ],
        out_specs=pl.BlockSpec((_R, HID), lambda i: (i, 0)),
        out_shape=jax.ShapeDtypeStruct((NP, HID), jnp.float32),
    )(x, W1, d0, d1)


def _tc2_body(p0_ref, p1_ref, y1_ref, d0_ref, d1_ref, b1_ref, w2_ref, y2_ref):
    dinv = _dinv_col(d0_ref[...], d1_ref[...])
    h = dinv * (p0_ref[...] + p1_ref[...] + y1_ref[...]) + b1_ref[...]
    h = jnp.maximum(h, 0.0)
    xw2 = jnp.dot(h, w2_ref[...], preferred_element_type=jnp.float32)
    y2_ref[...] = xw2 * dinv


def _tc_phase2(p0, p1, y1, d0, d1, b1, W2):
    return pl.pallas_call(
        _tc2_body,
        grid=(_GRID,),
        in_specs=[
            pl.BlockSpec((_R, HID), lambda i: (i, 0)),
            pl.BlockSpec((_R, HID), lambda i: (i, 0)),
            pl.BlockSpec((_R, HID), lambda i: (i, 0)),
            pl.BlockSpec((_R, DEG_D), lambda i: (i, 0)),
            pl.BlockSpec((_R, DEG_D), lambda i: (i, 0)),
            pl.BlockSpec((1, HID), lambda i: (0, 0)),
            pl.BlockSpec((HID, C), lambda i: (0, 0)),
        ],
        out_specs=pl.BlockSpec((_R, C), lambda i: (i, 0)),
        out_shape=jax.ShapeDtypeStruct((NP, C), jnp.float32),
    )(p0, p1, y1, d0, d1, b1, W2)


def _tc3_body(q0_ref, q1_ref, y2_ref, d0_ref, d1_ref, b2_ref, o_ref):
    dinv = _dinv_col(d0_ref[...], d1_ref[...])
    o = dinv * (q0_ref[...] + q1_ref[...] + y2_ref[...]) + b2_ref[...]
    m = jnp.max(o, axis=1, keepdims=True)
    e = jnp.exp(o - m)
    s = jnp.sum(e, axis=1, keepdims=True)
    o_ref[...] = o - m - jnp.log(s)


def _tc_phase3(q0, q1, y2, d0, d1, b2):
    return pl.pallas_call(
        _tc3_body,
        grid=(_GRID,),
        in_specs=[
            pl.BlockSpec((_R, C), lambda i: (i, 0)),
            pl.BlockSpec((_R, C), lambda i: (i, 0)),
            pl.BlockSpec((_R, C), lambda i: (i, 0)),
            pl.BlockSpec((_R, DEG_D), lambda i: (i, 0)),
            pl.BlockSpec((_R, DEG_D), lambda i: (i, 0)),
            pl.BlockSpec((1, C), lambda i: (0, 0)),
        ],
        out_specs=pl.BlockSpec((_R, C), lambda i: (i, 0)),
        out_shape=jax.ShapeDtypeStruct((N, C), jnp.float32),
    )(q0, q1, y2, d0, d1, b2)


def kernel(x, edge_index, W1, b1, W2, b2):
    src3 = edge_index[0].reshape(NW, NCH, K)
    dst3 = edge_index[1].reshape(NW, NCH, K)
    d0, d1 = _make_sc_degree()(edge_index)
    y1 = _tc_phase1(x, W1, d0, d1)
    p0, p1 = _make_sc_agg(HID, ())(y1, src3, dst3)
    y2 = _tc_phase2(p0, p1, y1, d0, d1, b1.reshape(1, HID), W2)
    q0, q1 = _make_sc_agg(C, (0, 1, 2, 3))(y2, src3, dst3)
    return _tc_phase3(q0, q1, y2, d0, d1, b2.reshape(1, C))
